# baseline (speedup 1.0000x reference)
import functools

import jax
import jax.numpy as jnp
from jax import lax
from jax.experimental import pallas as pl
from jax.experimental.pallas import tpu as pltpu
from jax.experimental.pallas import tpu_sc as plsc

VOCAB = 100000
SEQ = 2048
BATCH = 4
EMBED = 1024

NC = 2
NS = 16
L = 16
NW = NC * NS

TOKENS = BATCH * SEQ
TOK_PER_W = TOKENS // NW
K = 32
NCHUNK = TOK_PER_W // K
NVEC = EMBED // L
UNROLL = 4

_mesh = plsc.VectorSubcoreMesh(core_axis_name="c", subcore_axis_name="s")


def _xlane(v, idx):
    return lax.gather(
        v, idx[:, None],
        dimension_numbers=lax.GatherDimensionNumbers(
            offset_dims=(), collapsed_slice_dims=(0,), start_index_map=(0,)),
        slice_sizes=(1,),
        mode=lax.GatherScatterMode.PROMISE_IN_BOUNDS)


@functools.partial(
    pl.kernel,
    mesh=_mesh,
    out_type=jax.ShapeDtypeStruct((TOKENS, EMBED), jnp.float32),
    scratch_types=[
        pltpu.VMEM((K,), jnp.int32),
        pltpu.VMEM((K, EMBED), jnp.float32),
        pltpu.VMEM((K, EMBED), jnp.float32),
        pltpu.VMEM((EMBED,), jnp.float32),
        pltpu.VMEM((EMBED,), jnp.float32),
        pltpu.SemaphoreType.DMA,
    ],
)
def _sc_embed(ids_hbm, table_hbm, pos_hbm, gamma_hbm, beta_hbm, out_hbm,
              idx_v, emb_v, pos_v, g_v, b_v, sem):
    wid = lax.axis_index("s") * NC + lax.axis_index("c")
    base = wid * TOK_PER_W
    pos_base = (wid % (SEQ // TOK_PER_W)) * TOK_PER_W

    pltpu.sync_copy(gamma_hbm, g_v)
    pltpu.sync_copy(beta_hbm, b_v)

    inv_d = jnp.float32(1.0 / EMBED)
    zeros = jnp.zeros((L,), jnp.float32)

    for c in range(NCHUNK):
        off = c * K
        pltpu.sync_copy(ids_hbm.at[pl.ds(base + off, K)], idx_v)
        pltpu.sync_copy(pos_hbm.at[pl.ds(pos_base + off, K)], pos_v)
        pltpu.async_copy(table_hbm.at[idx_v], emb_v, sem).wait()

        def token_body(t, _, emb_v=emb_v, pos_v=pos_v):
            def p1(jj, accs):
                s, s2 = accs
                for u in range(UNROLL):
                    sl = pl.ds((jj * UNROLL + u) * L, L)
                    v = emb_v[t, sl] + pos_v[t, sl]
                    emb_v[t, sl] = v
                    s = s + v
                    s2 = s2 + v * v
                return (s, s2)

            s, s2 = lax.fori_loop(0, NVEC // UNROLL, p1, (zeros, zeros))
            for k in (8, 4, 2, 1):
                idx = lax.iota(jnp.int32, L) ^ k
                s = s + _xlane(s, idx)
                s2 = s2 + _xlane(s2, idx)
            mean_v = s * inv_d
            vv = s2 * inv_d - mean_v * mean_v + jnp.float32(1e-5)
            bits = lax.bitcast_convert_type(vv, jnp.int32)
            y = lax.bitcast_convert_type(
                jnp.int32(0x5F3759DF) - (bits >> 1), jnp.float32)
            for _ in range(3):
                y = y * (jnp.float32(1.5) - jnp.float32(0.5) * vv * y * y)

            def p2(jj, carry):
                for u in range(UNROLL):
                    sl = pl.ds((jj * UNROLL + u) * L, L)
                    emb_v[t, sl] = (emb_v[t, sl] - mean_v) * y * g_v[sl] + b_v[sl]
                return carry

            lax.fori_loop(0, NVEC // UNROLL, p2, 0)
            return 0

        lax.fori_loop(0, K, token_body, 0)
        pltpu.sync_copy(emb_v, out_hbm.at[pl.ds(base + off, K)])


def kernel(input_ids, token_table, pos_table, gamma, beta):
    ids_flat = input_ids.reshape(TOKENS).astype(jnp.int32)
    out = _sc_embed(ids_flat, token_table, pos_table, gamma, beta)
    return out.reshape(BATCH, SEQ, EMBED)
